# baseline (speedup 1.0000x reference)
import functools

import jax
import jax.numpy as jnp
from jax import lax
from jax.experimental import pallas as pl
from jax.experimental.pallas import tpu as pltpu
from jax.experimental.pallas import tpu_sc as plsc

N_GRAPHS = 64
NC = 2
NS = 16
NW = NC * NS
K = 128

_HIGH = jax.lax.Precision.HIGHEST


SB = 16


@functools.partial(jax.jit, static_argnames=("chunks", "n_pad", "h_dim"))
def _sc_spmm(h, src3, dst3, zeros, *, chunks, n_pad, h_dim):
  mesh = plsc.VectorSubcoreMesh(core_axis_name="c", subcore_axis_name="s")
  rpt = n_pad // NS

  @functools.partial(
      pl.kernel,
      out_type=jax.ShapeDtypeStruct((NC, n_pad, h_dim), jnp.float32),
      mesh=mesh,
      scratch_types=[
          pltpu.VMEM((chunks, K), jnp.int32),
          pltpu.VMEM((chunks, K), jnp.int32),
          pltpu.VMEM((K, h_dim), jnp.float32),
          pltpu.VMEM_SHARED((n_pad, h_dim), jnp.float32),
          pltpu.SemaphoreType.DMA,
      ],
  )
  def spmm(h_hbm, src_hbm, dst_hbm, z_hbm, out_hbm, src_v, dst_v, rows_v,
           acc, sem):
    c = lax.axis_index("c")
    s = lax.axis_index("s")
    w = c * NS + s
    pltpu.sync_copy(z_hbm.at[pl.ds(s * rpt, rpt)], acc.at[pl.ds(s * rpt, rpt)])
    pltpu.sync_copy(src_hbm.at[w], src_v)
    pltpu.sync_copy(dst_hbm.at[w], dst_v)
    plsc.subcore_barrier()

    def body(j, carry):
      pltpu.async_copy(h_hbm.at[src_v.at[j]], rows_v, sem).wait()
      pltpu.sync_copy(rows_v, acc.at[dst_v.at[j]], add=True)
      return carry

    lax.fori_loop(0, chunks, body, 0)
    plsc.subcore_barrier()
    pltpu.sync_copy(acc.at[pl.ds(s * rpt, rpt)],
                    out_hbm.at[c, pl.ds(s * rpt, rpt)])

  return spmm(h, src3, dst3, zeros)


@functools.partial(jax.jit, static_argnames=("chunks", "n_pad", "d", "h_dim"))
def _sc_eagg(e_attr, dst3, zeros, *, chunks, n_pad, d, h_dim):
  mesh = plsc.VectorSubcoreMesh(core_axis_name="c", subcore_axis_name="s")
  rpt = n_pad // NS

  per_row = h_dim // d
  kc = K // per_row

  @functools.partial(
      pl.kernel,
      out_type=jax.ShapeDtypeStruct((NC, n_pad, h_dim), jnp.float32),
      mesh=mesh,
      scratch_types=[
          pltpu.VMEM((chunks, K), jnp.int32),
          pltpu.VMEM((2, kc, h_dim), jnp.float32),
          pltpu.VMEM((K, h_dim), jnp.float32),
          pltpu.VMEM_SHARED((n_pad, h_dim), jnp.float32),
          pltpu.SemaphoreType.DMA((2,)),
      ],
  )
  def eagg(ea_hbm, dst_hbm, z_hbm, out_hbm, dst_v, ebuf_c, ebuf, acc, sems):
    c = lax.axis_index("c")
    s = lax.axis_index("s")
    w = c * NS + s
    pltpu.sync_copy(z_hbm.at[pl.ds(s * rpt, rpt)], acc.at[pl.ds(s * rpt, rpt)])
    pltpu.sync_copy(dst_hbm.at[w], dst_v)
    pltpu.sync_copy(z_hbm.at[pl.ds(0, K)], ebuf)
    plsc.subcore_barrier()
    base_c = w * (chunks * kc)

    pltpu.async_copy(ea_hbm.at[pl.ds(base_c, kc)], ebuf_c.at[0], sems.at[0])

    def body(j, carry):
      bi = lax.rem(j, 2)
      ni = lax.rem(j + 1, 2)

      @pl.when(j + 1 < chunks)
      def _():
        pltpu.async_copy(ea_hbm.at[pl.ds(base_c + (j + 1) * kc, kc)],
                         ebuf_c.at[ni], sems.at[ni])

      pltpu.make_async_copy(ea_hbm.at[pl.ds(base_c, kc)], ebuf_c.at[bi],
                            sems.at[bi]).wait()

      def expand(r, carry2):
        for g in range(per_row):
          ebuf[r * per_row + g, pl.ds(0, d)] = ebuf_c[bi, r, pl.ds(g * d, d)]
        return carry2

      lax.fori_loop(0, kc, expand, 0)
      pltpu.sync_copy(ebuf, acc.at[dst_v.at[j]], add=True)
      return carry

    lax.fori_loop(0, chunks, body, 0)
    plsc.subcore_barrier()
    pltpu.sync_copy(acc.at[pl.ds(s * rpt, rpt)],
                    out_hbm.at[c, pl.ds(s * rpt, rpt)])

  return eagg(e_attr, dst3, zeros)


def _tc_prep(x, emb1, emb2, eagg, w_edge, *, bn, nb):
  def body(x_ref, e1_ref, e2_ref, eg_ref, we_ref, h_ref, ep_ref):
    x0 = x_ref[:, 0:1]
    x1 = x_ref[:, 1:2]
    e1 = e1_ref[...]
    e2 = e2_ref[...]
    h = jnp.where(x0 == 0, e1[0:1, :], jnp.where(x0 == 1, e1[1:2, :],
                                                 e1[2:3, :]))
    h = h + jnp.where(x1 == 0, e2[0:1, :], jnp.where(x1 == 1, e2[1:2, :],
                                                     e2[2:3, :]))
    d = we_ref.shape[0]
    ep = eg_ref[0, :, :d] + eg_ref[1, :, :d]
    h_ref[...] = h
    we = we_ref[...].astype(jnp.bfloat16).astype(jnp.float32)
    ep_ref[...] = jnp.dot(ep, we, precision=_HIGH,
                          preferred_element_type=jnp.float32)

  n, hd = x.shape[0], emb1.shape[1]
  d = w_edge.shape[0]
  return pl.pallas_call(
      body,
      grid=(nb,),
      in_specs=[
          pl.BlockSpec((bn, 2), lambda i: (i, 0)),
          pl.BlockSpec(emb1.shape, lambda i: (0, 0)),
          pl.BlockSpec(emb2.shape, lambda i: (0, 0)),
          pl.BlockSpec((NC, bn, hd), lambda i: (0, i, 0)),
          pl.BlockSpec((d, hd), lambda i: (0, 0)),
      ],
      out_specs=[
          pl.BlockSpec((bn, hd), lambda i: (i, 0)),
          pl.BlockSpec((bn, hd), lambda i: (i, 0)),
      ],
      out_shape=[
          jax.ShapeDtypeStruct((n, hd), jnp.float32),
          jax.ShapeDtypeStruct((n, hd), jnp.float32),
      ],
  )(x, emb1, emb2, eagg, w_edge)


def _tc_layer_mlp(h, s_part, eproj, w1, b1, w2, b2, *, bn, nb):
  def body(h_ref, s_ref, ep_ref, w1_ref, b1_ref, w2_ref, b2_ref, z_ref,
           st_ref):
    i = pl.program_id(0)
    hin = h_ref[...] + s_ref[0] + s_ref[1] + ep_ref[...]
    a = jnp.maximum(
        jnp.dot(hin.astype(jnp.bfloat16), w1_ref[...].astype(jnp.bfloat16),
                preferred_element_type=jnp.float32) + b1_ref[...], 0.0)
    z = jnp.dot(a.astype(jnp.bfloat16), w2_ref[...].astype(jnp.bfloat16),
                preferred_element_type=jnp.float32) + b2_ref[...]
    z_ref[...] = z

    @pl.when(i == 0)
    def _():
      st_ref[0:2, :] = jnp.zeros_like(st_ref[0:2, :])
      st_ref[2:3, :] = jnp.mean(z, axis=0, keepdims=True)

    m0 = st_ref[2:3, :]
    zc = z - m0
    st_ref[0:1, :] += jnp.sum(zc, axis=0, keepdims=True)
    st_ref[1:2, :] += jnp.sum(zc * zc, axis=0, keepdims=True)

  n, hd = h.shape
  h2 = w1.shape[1]
  return pl.pallas_call(
      body,
      grid=(nb,),
      in_specs=[
          pl.BlockSpec((bn, hd), lambda i: (i, 0)),
          pl.BlockSpec((NC, bn, hd), lambda i: (0, i, 0)),
          pl.BlockSpec((bn, hd), lambda i: (i, 0)),
          pl.BlockSpec((hd, h2), lambda i: (0, 0)),
          pl.BlockSpec((1, h2), lambda i: (0, 0)),
          pl.BlockSpec((h2, hd), lambda i: (0, 0)),
          pl.BlockSpec((1, hd), lambda i: (0, 0)),
      ],
      out_specs=[
          pl.BlockSpec((bn, hd), lambda i: (i, 0)),
          pl.BlockSpec((8, hd), lambda i: (0, 0)),
      ],
      out_shape=[
          jax.ShapeDtypeStruct((n, hd), jnp.float32),
          jax.ShapeDtypeStruct((8, hd), jnp.float32),
      ],
  )(h, s_part, eproj, w1, b1, w2, b2)


def _tc_bn(z, stats, gamma, beta, *, relu, n, bn, nb):
  inv_n = 1.0 / n

  def body(z_ref, st_ref, g_ref, b_ref, o_ref):
    dm = st_ref[0:1, :] * inv_n
    mean = st_ref[2:3, :] + dm
    var = st_ref[1:2, :] * inv_n - dm * dm
    rstd = lax.rsqrt(var + 1e-5)
    y = (z_ref[...] - mean) * (rstd * g_ref[...]) + b_ref[...]
    if relu:
      y = jnp.maximum(y, 0.0)
    o_ref[...] = y

  hd = z.shape[1]
  return pl.pallas_call(
      body,
      grid=(nb,),
      in_specs=[
          pl.BlockSpec((bn, hd), lambda i: (i, 0)),
          pl.BlockSpec((8, hd), lambda i: (0, 0)),
          pl.BlockSpec((1, hd), lambda i: (0, 0)),
          pl.BlockSpec((1, hd), lambda i: (0, 0)),
      ],
      out_specs=pl.BlockSpec((bn, hd), lambda i: (i, 0)),
      out_shape=jax.ShapeDtypeStruct(z.shape, jnp.float32),
  )(z, stats, gamma, beta)


def _tc_pool(h, batch2d, w_feat, b_feat, w_out_p, b_out_p, *, bn, nb):
  def body(h_ref, bt_ref, wf_ref, bf_ref, wo_ref, bo_ref, g_ref, l_ref,
           gsum, cnt):
    i = pl.program_id(0)

    @pl.when(i == 0)
    def _():
      gsum[...] = jnp.zeros_like(gsum)
      cnt[...] = jnp.zeros_like(cnt)

    oh = (lax.broadcasted_iota(jnp.int32, (bn, N_GRAPHS), 1)
          == bt_ref[...]).astype(jnp.float32)
    dn = (((0,), (0,)), ((), ()))
    gsum[...] += lax.dot_general(oh, h_ref[...], dn, precision=_HIGH,
                                 preferred_element_type=jnp.float32)
    cnt[...] += lax.dot_general(oh, jnp.ones_like(h_ref[...]), dn,
                                precision=_HIGH,
                                preferred_element_type=jnp.float32)

    @pl.when(i == nb - 1)
    def _():
      g = gsum[...] / jnp.maximum(cnt[...], 1.0)
      go = jnp.dot(g.astype(jnp.bfloat16), wf_ref[...].astype(jnp.bfloat16),
                   preferred_element_type=jnp.float32) + bf_ref[...]
      g_ref[...] = go
      l_ref[...] = jnp.dot(go.astype(jnp.bfloat16),
                           wo_ref[...].astype(jnp.bfloat16),
                           preferred_element_type=jnp.float32) + bo_ref[...]

  hd = h.shape[1]
  return pl.pallas_call(
      body,
      grid=(nb,),
      in_specs=[
          pl.BlockSpec((bn, hd), lambda i: (i, 0)),
          pl.BlockSpec((bn, 1), lambda i: (i, 0)),
          pl.BlockSpec((hd, hd), lambda i: (0, 0)),
          pl.BlockSpec((1, hd), lambda i: (0, 0)),
          pl.BlockSpec((hd, hd), lambda i: (0, 0)),
          pl.BlockSpec((1, hd), lambda i: (0, 0)),
      ],
      out_specs=[
          pl.BlockSpec((N_GRAPHS, hd), lambda i: (0, 0)),
          pl.BlockSpec((N_GRAPHS, hd), lambda i: (0, 0)),
      ],
      out_shape=[
          jax.ShapeDtypeStruct((N_GRAPHS, hd), jnp.float32),
          jax.ShapeDtypeStruct((N_GRAPHS, hd), jnp.float32),
      ],
      scratch_shapes=[
          pltpu.VMEM((N_GRAPHS, hd), jnp.float32),
          pltpu.VMEM((N_GRAPHS, hd), jnp.float32),
      ],
  )(h, batch2d, w_feat, b_feat, w_out_p, b_out_p)


def kernel(x, e_index, e_attr, batch, emb1, emb2, W_edge, W1, b1, W2, b2,
           bn_scale, bn_bias, W_feat, b_feat, W_out, b_out):
  n, hd = x.shape[0], emb1.shape[1]
  e = e_index.shape[1]
  d = e_attr.shape[1]
  num_layers = W1.shape[0]
  n_classes = W_out.shape[1]
  bn = 1000
  nb = n // bn

  chunks = SB * (-(-e // (NW * K * SB)))
  e_pad = NW * chunks * K
  pad = e_pad - e
  n_pad = NS * 8 * (-(-(n + 1) // (NS * 8)))

  src = e_index[0].astype(jnp.int32)
  dst = e_index[1].astype(jnp.int32)
  src3 = jnp.concatenate([src, jnp.zeros((pad,), jnp.int32)]).reshape(
      NW, chunks, K)
  trash = n + jnp.arange(pad, dtype=jnp.int32) % (n_pad - n)
  dst3 = jnp.concatenate([dst, trash]).reshape(NW, chunks, K)
  ea_bf = e_attr.astype(jnp.bfloat16).astype(jnp.float32)
  ea_c = jnp.concatenate([ea_bf, jnp.zeros((pad, d), jnp.float32)]).reshape(
      e_pad * d // hd, hd)
  zeros_h = jnp.zeros((n_pad, hd), jnp.float32)

  eagg = _sc_eagg(ea_c, dst3, zeros_h, chunks=chunks, n_pad=n_pad, d=d,
                  h_dim=hd)
  h, eproj = _tc_prep(x.astype(jnp.int32), emb1, emb2, eagg, W_edge,
                      bn=bn, nb=nb)

  b1r = b1.reshape(num_layers, 1, -1)
  b2r = b2.reshape(num_layers, 1, -1)
  for l in range(num_layers):
    s_part = _sc_spmm(h, src3, dst3, zeros_h, chunks=chunks, n_pad=n_pad,
                      h_dim=hd)
    z, stats = _tc_layer_mlp(h, s_part, eproj, W1[l], b1r[l], W2[l], b2r[l],
                             bn=bn, nb=nb)
    h = _tc_bn(z, stats, bn_scale[l:l + 1], bn_bias[l:l + 1],
               relu=(l < num_layers - 1), n=n, bn=bn, nb=nb)

  w_out_p = jnp.zeros((hd, hd), jnp.float32).at[:, :n_classes].set(W_out)
  b_out_p = jnp.zeros((1, hd), jnp.float32).at[0, :n_classes].set(b_out)
  g, logits_p = _tc_pool(h, batch.astype(jnp.int32).reshape(n, 1),
                         W_feat, b_feat.reshape(1, hd), w_out_p, b_out_p,
                         bn=bn, nb=nb)
  return (g, logits_p[:, :n_classes])


# speedup vs baseline: 2.1104x; 2.1104x over previous
import functools

import jax
import jax.numpy as jnp
from jax import lax
from jax.experimental import pallas as pl
from jax.experimental.pallas import tpu as pltpu
from jax.experimental.pallas import tpu_sc as plsc

N_GRAPHS = 64
NC = 2
NS = 16
NW = NC * NS
K = 128

_HIGH = jax.lax.Precision.HIGHEST


SB = 16


@functools.partial(jax.jit, static_argnames=("chunks", "n_pad", "h_dim"))
def _sc_spmm(h, src3, dst3, zeros, *, chunks, n_pad, h_dim):
  mesh = plsc.VectorSubcoreMesh(core_axis_name="c", subcore_axis_name="s")
  rpt = n_pad // NS

  @functools.partial(
      pl.kernel,
      out_type=jax.ShapeDtypeStruct((NC, n_pad, h_dim), jnp.float32),
      mesh=mesh,
      scratch_types=[
          pltpu.VMEM((chunks, K), jnp.int32),
          pltpu.VMEM((chunks, K), jnp.int32),
          pltpu.VMEM((K, h_dim), jnp.float32),
          pltpu.VMEM_SHARED((n_pad, h_dim), jnp.float32),
          pltpu.SemaphoreType.DMA,
      ],
  )
  def spmm(h_hbm, src_hbm, dst_hbm, z_hbm, out_hbm, src_v, dst_v, rows_v,
           acc, sem):
    c = lax.axis_index("c")
    s = lax.axis_index("s")
    w = c * NS + s
    pltpu.sync_copy(z_hbm.at[pl.ds(s * rpt, rpt)], acc.at[pl.ds(s * rpt, rpt)])
    pltpu.sync_copy(src_hbm.at[w], src_v)
    pltpu.sync_copy(dst_hbm.at[w], dst_v)
    plsc.subcore_barrier()

    def body(j, carry):
      pltpu.async_copy(h_hbm.at[src_v.at[j]], rows_v, sem).wait()
      pltpu.sync_copy(rows_v, acc.at[dst_v.at[j]], add=True)
      return carry

    lax.fori_loop(0, chunks, body, 0)
    plsc.subcore_barrier()
    pltpu.sync_copy(acc.at[pl.ds(s * rpt, rpt)],
                    out_hbm.at[c, pl.ds(s * rpt, rpt)])

  return spmm(h, src3, dst3, zeros)


@functools.partial(jax.jit, static_argnames=("chunks", "n_pad", "d", "h_dim"))
def _sc_eagg(e_attr, dst3, zeros, *, chunks, n_pad, d, h_dim):
  mesh = plsc.VectorSubcoreMesh(core_axis_name="c", subcore_axis_name="s")
  rpt = n_pad // NS

  per_row = h_dim // d
  kc = K // per_row

  @functools.partial(
      pl.kernel,
      out_type=jax.ShapeDtypeStruct((NC, n_pad, h_dim), jnp.float32),
      mesh=mesh,
      scratch_types=[
          pltpu.VMEM((chunks, K), jnp.int32),
          pltpu.VMEM((2, kc, h_dim), jnp.float32),
          pltpu.VMEM((K, h_dim), jnp.float32),
          pltpu.VMEM_SHARED((n_pad, h_dim), jnp.float32),
          pltpu.SemaphoreType.DMA((2,)),
      ],
  )
  def eagg(ea_hbm, dst_hbm, z_hbm, out_hbm, dst_v, ebuf_c, ebuf, acc, sems):
    c = lax.axis_index("c")
    s = lax.axis_index("s")
    w = c * NS + s
    pltpu.sync_copy(z_hbm.at[pl.ds(s * rpt, rpt)], acc.at[pl.ds(s * rpt, rpt)])
    pltpu.sync_copy(dst_hbm.at[w], dst_v)
    pltpu.sync_copy(z_hbm.at[pl.ds(0, K)], ebuf)
    plsc.subcore_barrier()
    base_c = w * (chunks * kc)

    pltpu.async_copy(ea_hbm.at[pl.ds(base_c, kc)], ebuf_c.at[0], sems.at[0])

    def body(j, carry):
      bi = lax.rem(j, 2)
      ni = lax.rem(j + 1, 2)

      @pl.when(j + 1 < chunks)
      def _():
        pltpu.async_copy(ea_hbm.at[pl.ds(base_c + (j + 1) * kc, kc)],
                         ebuf_c.at[ni], sems.at[ni])

      pltpu.make_async_copy(ea_hbm.at[pl.ds(base_c, kc)], ebuf_c.at[bi],
                            sems.at[bi]).wait()

      def expand(r, carry2):
        for g in range(per_row):
          ebuf[r * per_row + g, pl.ds(0, d)] = ebuf_c[bi, r, pl.ds(g * d, d)]
        return carry2

      lax.fori_loop(0, kc, expand, 0)
      pltpu.sync_copy(ebuf, acc.at[dst_v.at[j]], add=True)
      return carry

    lax.fori_loop(0, chunks, body, 0)
    plsc.subcore_barrier()
    pltpu.sync_copy(acc.at[pl.ds(s * rpt, rpt)],
                    out_hbm.at[c, pl.ds(s * rpt, rpt)])

  return eagg(e_attr, dst3, zeros)


def _tc_prep(x, emb1, emb2, eagg, w_edge, *, bn, nb):
  def body(x_ref, e1_ref, e2_ref, eg_ref, we_ref, h_ref, ep_ref):
    x0 = x_ref[:, 0:1]
    x1 = x_ref[:, 1:2]
    e1 = e1_ref[...]
    e2 = e2_ref[...]
    h = jnp.where(x0 == 0, e1[0:1, :], jnp.where(x0 == 1, e1[1:2, :],
                                                 e1[2:3, :]))
    h = h + jnp.where(x1 == 0, e2[0:1, :], jnp.where(x1 == 1, e2[1:2, :],
                                                     e2[2:3, :]))
    d = we_ref.shape[0]
    ep = eg_ref[0, :, :d] + eg_ref[1, :, :d]
    h_ref[...] = h
    we = we_ref[...].astype(jnp.bfloat16).astype(jnp.float32)
    ep_ref[...] = jnp.dot(ep, we, precision=_HIGH,
                          preferred_element_type=jnp.float32)

  n, hd = x.shape[0], emb1.shape[1]
  d = w_edge.shape[0]
  return pl.pallas_call(
      body,
      grid=(nb,),
      in_specs=[
          pl.BlockSpec((bn, 2), lambda i: (i, 0)),
          pl.BlockSpec(emb1.shape, lambda i: (0, 0)),
          pl.BlockSpec(emb2.shape, lambda i: (0, 0)),
          pl.BlockSpec((NC, bn, hd), lambda i: (0, i, 0)),
          pl.BlockSpec((d, hd), lambda i: (0, 0)),
      ],
      out_specs=[
          pl.BlockSpec((bn, hd), lambda i: (i, 0)),
          pl.BlockSpec((bn, hd), lambda i: (i, 0)),
      ],
      out_shape=[
          jax.ShapeDtypeStruct((n, hd), jnp.float32),
          jax.ShapeDtypeStruct((n, hd), jnp.float32),
      ],
  )(x, emb1, emb2, eagg, w_edge)


def _tc_layer_mlp(h, s_part, eproj, w1, b1, w2, b2, *, bn, nb):
  def body(h_ref, s_ref, ep_ref, w1_ref, b1_ref, w2_ref, b2_ref, z_ref,
           st_ref):
    i = pl.program_id(0)
    hin = h_ref[...] + s_ref[0] + s_ref[1] + ep_ref[...]
    a = jnp.maximum(
        jnp.dot(hin.astype(jnp.bfloat16), w1_ref[...].astype(jnp.bfloat16),
                preferred_element_type=jnp.float32) + b1_ref[...], 0.0)
    z = jnp.dot(a.astype(jnp.bfloat16), w2_ref[...].astype(jnp.bfloat16),
                preferred_element_type=jnp.float32) + b2_ref[...]
    z_ref[...] = z

    @pl.when(i == 0)
    def _():
      st_ref[0:2, :] = jnp.zeros_like(st_ref[0:2, :])
      st_ref[2:3, :] = jnp.mean(z, axis=0, keepdims=True)

    m0 = st_ref[2:3, :]
    zc = z - m0
    st_ref[0:1, :] += jnp.sum(zc, axis=0, keepdims=True)
    st_ref[1:2, :] += jnp.sum(zc * zc, axis=0, keepdims=True)

  n, hd = h.shape
  h2 = w1.shape[1]
  return pl.pallas_call(
      body,
      grid=(nb,),
      in_specs=[
          pl.BlockSpec((bn, hd), lambda i: (i, 0)),
          pl.BlockSpec((NC, bn, hd), lambda i: (0, i, 0)),
          pl.BlockSpec((bn, hd), lambda i: (i, 0)),
          pl.BlockSpec((hd, h2), lambda i: (0, 0)),
          pl.BlockSpec((1, h2), lambda i: (0, 0)),
          pl.BlockSpec((h2, hd), lambda i: (0, 0)),
          pl.BlockSpec((1, hd), lambda i: (0, 0)),
      ],
      out_specs=[
          pl.BlockSpec((bn, hd), lambda i: (i, 0)),
          pl.BlockSpec((8, hd), lambda i: (0, 0)),
      ],
      out_shape=[
          jax.ShapeDtypeStruct((n, hd), jnp.float32),
          jax.ShapeDtypeStruct((8, hd), jnp.float32),
      ],
  )(h, s_part, eproj, w1, b1, w2, b2)


def _tc_bn(z, stats, gamma, beta, *, relu, n, bn, nb):
  inv_n = 1.0 / n

  def body(z_ref, st_ref, g_ref, b_ref, o_ref):
    dm = st_ref[0:1, :] * inv_n
    mean = st_ref[2:3, :] + dm
    var = st_ref[1:2, :] * inv_n - dm * dm
    rstd = lax.rsqrt(var + 1e-5)
    y = (z_ref[...] - mean) * (rstd * g_ref[...]) + b_ref[...]
    if relu:
      y = jnp.maximum(y, 0.0)
    o_ref[...] = y

  hd = z.shape[1]
  return pl.pallas_call(
      body,
      grid=(nb,),
      in_specs=[
          pl.BlockSpec((bn, hd), lambda i: (i, 0)),
          pl.BlockSpec((8, hd), lambda i: (0, 0)),
          pl.BlockSpec((1, hd), lambda i: (0, 0)),
          pl.BlockSpec((1, hd), lambda i: (0, 0)),
      ],
      out_specs=pl.BlockSpec((bn, hd), lambda i: (i, 0)),
      out_shape=jax.ShapeDtypeStruct(z.shape, jnp.float32),
  )(z, stats, gamma, beta)


def _tc_pool(h, batch2d, w_feat, b_feat, w_out_p, b_out_p, *, bn, nb):
  def body(h_ref, bt_ref, wf_ref, bf_ref, wo_ref, bo_ref, g_ref, l_ref,
           gsum, cnt):
    i = pl.program_id(0)

    @pl.when(i == 0)
    def _():
      gsum[...] = jnp.zeros_like(gsum)
      cnt[...] = jnp.zeros_like(cnt)

    oh = (lax.broadcasted_iota(jnp.int32, (bn, N_GRAPHS), 1)
          == bt_ref[...]).astype(jnp.float32)
    dn = (((0,), (0,)), ((), ()))
    gsum[...] += lax.dot_general(oh, h_ref[...], dn, precision=_HIGH,
                                 preferred_element_type=jnp.float32)
    cnt[...] += lax.dot_general(oh, jnp.ones_like(h_ref[...]), dn,
                                precision=_HIGH,
                                preferred_element_type=jnp.float32)

    @pl.when(i == nb - 1)
    def _():
      g = gsum[...] / jnp.maximum(cnt[...], 1.0)
      go = jnp.dot(g.astype(jnp.bfloat16), wf_ref[...].astype(jnp.bfloat16),
                   preferred_element_type=jnp.float32) + bf_ref[...]
      g_ref[...] = go
      l_ref[...] = jnp.dot(go.astype(jnp.bfloat16),
                           wo_ref[...].astype(jnp.bfloat16),
                           preferred_element_type=jnp.float32) + bo_ref[...]

  hd = h.shape[1]
  return pl.pallas_call(
      body,
      grid=(nb,),
      in_specs=[
          pl.BlockSpec((bn, hd), lambda i: (i, 0)),
          pl.BlockSpec((bn, 1), lambda i: (i, 0)),
          pl.BlockSpec((hd, hd), lambda i: (0, 0)),
          pl.BlockSpec((1, hd), lambda i: (0, 0)),
          pl.BlockSpec((hd, hd), lambda i: (0, 0)),
          pl.BlockSpec((1, hd), lambda i: (0, 0)),
      ],
      out_specs=[
          pl.BlockSpec((N_GRAPHS, hd), lambda i: (0, 0)),
          pl.BlockSpec((N_GRAPHS, hd), lambda i: (0, 0)),
      ],
      out_shape=[
          jax.ShapeDtypeStruct((N_GRAPHS, hd), jnp.float32),
          jax.ShapeDtypeStruct((N_GRAPHS, hd), jnp.float32),
      ],
      scratch_shapes=[
          pltpu.VMEM((N_GRAPHS, hd), jnp.float32),
          pltpu.VMEM((N_GRAPHS, hd), jnp.float32),
      ],
  )(h, batch2d, w_feat, b_feat, w_out_p, b_out_p)


def kernel(x, e_index, e_attr, batch, emb1, emb2, W_edge, W1, b1, W2, b2,
           bn_scale, bn_bias, W_feat, b_feat, W_out, b_out):
  n, hd = x.shape[0], emb1.shape[1]
  e = e_index.shape[1]
  d = e_attr.shape[1]
  num_layers = W1.shape[0]
  n_classes = W_out.shape[1]
  bn = 1000
  nb = n // bn

  chunks = SB * (-(-e // (NW * K * SB)))
  e_pad = NW * chunks * K
  pad = e_pad - e
  n_pad = NS * 8 * (-(-(n + 1) // (NS * 8)))

  src = e_index[0].astype(jnp.int32)
  dst = e_index[1].astype(jnp.int32)
  pad_src = jnp.arange(pad, dtype=jnp.int32) % n
  src3 = jnp.concatenate([src, pad_src]).reshape(NW, chunks, K)
  trash = n + jnp.arange(pad, dtype=jnp.int32) % (n_pad - n)
  dst3 = jnp.concatenate([dst, trash]).reshape(NW, chunks, K)
  ea_bf = e_attr.astype(jnp.bfloat16).astype(jnp.float32)
  ea_c = jnp.concatenate([ea_bf, jnp.zeros((pad, d), jnp.float32)]).reshape(
      e_pad * d // hd, hd)
  zeros_h = jnp.zeros((n_pad, hd), jnp.float32)

  eagg = _sc_eagg(ea_c, dst3, zeros_h, chunks=chunks, n_pad=n_pad, d=d,
                  h_dim=hd)
  h, eproj = _tc_prep(x.astype(jnp.int32), emb1, emb2, eagg, W_edge,
                      bn=bn, nb=nb)

  b1r = b1.reshape(num_layers, 1, -1)
  b2r = b2.reshape(num_layers, 1, -1)
  for l in range(num_layers):
    s_part = _sc_spmm(h, src3, dst3, zeros_h, chunks=chunks, n_pad=n_pad,
                      h_dim=hd)
    z, stats = _tc_layer_mlp(h, s_part, eproj, W1[l], b1r[l], W2[l], b2r[l],
                             bn=bn, nb=nb)
    h = _tc_bn(z, stats, bn_scale[l:l + 1], bn_bias[l:l + 1],
               relu=(l < num_layers - 1), n=n, bn=bn, nb=nb)

  w_out_p = jnp.zeros((hd, hd), jnp.float32).at[:, :n_classes].set(W_out)
  b_out_p = jnp.zeros((1, hd), jnp.float32).at[0, :n_classes].set(b_out)
  g, logits_p = _tc_pool(h, batch.astype(jnp.int32).reshape(n, 1),
                         W_feat, b_feat.reshape(1, hd), w_out_p, b_out_p,
                         bn=bn, nb=nb)
  return (g, logits_p[:, :n_classes])


# speedup vs baseline: 2.6913x; 1.2753x over previous
import functools

import jax
import jax.numpy as jnp
from jax import lax
from jax.experimental import pallas as pl
from jax.experimental.pallas import tpu as pltpu
from jax.experimental.pallas import tpu_sc as plsc

N_GRAPHS = 64
NC = 2
NS = 16
NW = NC * NS
K = 128

_HIGH = jax.lax.Precision.HIGHEST


SB = 16


@functools.partial(jax.jit, static_argnames=("chunks", "n_pad", "h_dim"))
def _sc_spmm(h, src3, dst3, zeros, *, chunks, n_pad, h_dim):
  mesh = plsc.VectorSubcoreMesh(core_axis_name="c", subcore_axis_name="s")
  rpt = n_pad // NS
  supers = chunks // SB

  @functools.partial(
      pl.kernel,
      out_type=jax.ShapeDtypeStruct((NC, n_pad, h_dim), jnp.float32),
      mesh=mesh,
      scratch_types=[
          pltpu.VMEM((2, SB, K), jnp.int32),
          pltpu.VMEM((2, SB, K), jnp.int32),
          pltpu.VMEM((2, K, h_dim), jnp.float32),
          pltpu.VMEM_SHARED((n_pad, h_dim), jnp.float32),
          pltpu.SemaphoreType.DMA((2,)),
          pltpu.SemaphoreType.DMA((2, 2)),
      ],
  )
  def spmm(h_hbm, src_hbm, dst_hbm, z_hbm, out_hbm, src_sv, dst_sv, rows_v,
           acc, gsems, isems):
    c = lax.axis_index("c")
    s = lax.axis_index("s")
    w = c * NS + s
    pltpu.async_copy(src_hbm.at[w, pl.ds(0, SB)], src_sv.at[0],
                     isems.at[0, 0])
    pltpu.async_copy(dst_hbm.at[w, pl.ds(0, SB)], dst_sv.at[0],
                     isems.at[0, 1])
    pltpu.sync_copy(z_hbm.at[pl.ds(s * rpt, rpt)], acc.at[pl.ds(s * rpt, rpt)])
    plsc.subcore_barrier()

    def outer(sb, carry):
      ib = lax.rem(sb, 2)
      nib = lax.rem(sb + 1, 2)
      pltpu.make_async_copy(src_hbm.at[w, pl.ds(0, SB)], src_sv.at[ib],
                            isems.at[ib, 0]).wait()
      pltpu.make_async_copy(dst_hbm.at[w, pl.ds(0, SB)], dst_sv.at[ib],
                            isems.at[ib, 1]).wait()

      @pl.when(sb + 1 < supers)
      def _():
        pltpu.async_copy(src_hbm.at[w, pl.ds((sb + 1) * SB, SB)],
                         src_sv.at[nib], isems.at[nib, 0])
        pltpu.async_copy(dst_hbm.at[w, pl.ds((sb + 1) * SB, SB)],
                         dst_sv.at[nib], isems.at[nib, 1])

      pltpu.async_copy(h_hbm.at[src_sv.at[ib, 0]], rows_v.at[0],
                       gsems.at[0])

      def inner(jo, carry2):
        bi = lax.rem(jo, 2)
        nbi = lax.rem(jo + 1, 2)

        @pl.when(jo + 1 < SB)
        def _():
          pltpu.async_copy(h_hbm.at[src_sv.at[ib, jo + 1]], rows_v.at[nbi],
                           gsems.at[nbi])

        pltpu.make_async_copy(h_hbm.at[src_sv.at[ib, 0]], rows_v.at[bi],
                              gsems.at[bi]).wait()
        pltpu.sync_copy(rows_v.at[bi], acc.at[dst_sv.at[ib, jo]], add=True)
        return carry2

      lax.fori_loop(0, SB, inner, 0)
      return carry

    lax.fori_loop(0, supers, outer, 0)
    plsc.subcore_barrier()
    pltpu.sync_copy(acc.at[pl.ds(s * rpt, rpt)],
                    out_hbm.at[c, pl.ds(s * rpt, rpt)])

  return spmm(h, src3, dst3, zeros)


@functools.partial(jax.jit, static_argnames=("chunks", "n_pad", "d", "h_dim"))
def _sc_eagg(e_attr, dst3, zeros, *, chunks, n_pad, d, h_dim):
  mesh = plsc.VectorSubcoreMesh(core_axis_name="c", subcore_axis_name="s")
  rpt = n_pad // NS

  per_row = h_dim // d
  kc = K // per_row

  @functools.partial(
      pl.kernel,
      out_type=jax.ShapeDtypeStruct((NC, n_pad, h_dim), jnp.float32),
      mesh=mesh,
      scratch_types=[
          pltpu.VMEM((chunks, K), jnp.int32),
          pltpu.VMEM((2, kc, h_dim), jnp.float32),
          pltpu.VMEM((K, h_dim), jnp.float32),
          pltpu.VMEM_SHARED((n_pad, h_dim), jnp.float32),
          pltpu.SemaphoreType.DMA((2,)),
      ],
  )
  def eagg(ea_hbm, dst_hbm, z_hbm, out_hbm, dst_v, ebuf_c, ebuf, acc, sems):
    c = lax.axis_index("c")
    s = lax.axis_index("s")
    w = c * NS + s
    pltpu.sync_copy(z_hbm.at[pl.ds(s * rpt, rpt)], acc.at[pl.ds(s * rpt, rpt)])
    pltpu.sync_copy(dst_hbm.at[w], dst_v)
    pltpu.sync_copy(z_hbm.at[pl.ds(0, K)], ebuf)
    plsc.subcore_barrier()
    base_c = w * (chunks * kc)

    pltpu.async_copy(ea_hbm.at[pl.ds(base_c, kc)], ebuf_c.at[0], sems.at[0])

    def body(j, carry):
      bi = lax.rem(j, 2)
      ni = lax.rem(j + 1, 2)

      @pl.when(j + 1 < chunks)
      def _():
        pltpu.async_copy(ea_hbm.at[pl.ds(base_c + (j + 1) * kc, kc)],
                         ebuf_c.at[ni], sems.at[ni])

      pltpu.make_async_copy(ea_hbm.at[pl.ds(base_c, kc)], ebuf_c.at[bi],
                            sems.at[bi]).wait()

      def expand(r, carry2):
        for g in range(per_row):
          ebuf[r * per_row + g, pl.ds(0, d)] = ebuf_c[bi, r, pl.ds(g * d, d)]
        return carry2

      lax.fori_loop(0, kc, expand, 0)
      pltpu.sync_copy(ebuf, acc.at[dst_v.at[j]], add=True)
      return carry

    lax.fori_loop(0, chunks, body, 0)
    plsc.subcore_barrier()
    pltpu.sync_copy(acc.at[pl.ds(s * rpt, rpt)],
                    out_hbm.at[c, pl.ds(s * rpt, rpt)])

  return eagg(e_attr, dst3, zeros)


def _tc_prep(x, emb1, emb2, eagg, w_edge, *, bn, nb):
  def body(x_ref, e1_ref, e2_ref, eg_ref, we_ref, h_ref, ep_ref):
    x0 = x_ref[:, 0:1]
    x1 = x_ref[:, 1:2]
    e1 = e1_ref[...]
    e2 = e2_ref[...]
    h = jnp.where(x0 == 0, e1[0:1, :], jnp.where(x0 == 1, e1[1:2, :],
                                                 e1[2:3, :]))
    h = h + jnp.where(x1 == 0, e2[0:1, :], jnp.where(x1 == 1, e2[1:2, :],
                                                     e2[2:3, :]))
    d = we_ref.shape[0]
    ep = eg_ref[0, :, :d] + eg_ref[1, :, :d]
    h_ref[...] = h
    we = we_ref[...].astype(jnp.bfloat16).astype(jnp.float32)
    ep_ref[...] = jnp.dot(ep, we, precision=_HIGH,
                          preferred_element_type=jnp.float32)

  n, hd = x.shape[0], emb1.shape[1]
  d = w_edge.shape[0]
  return pl.pallas_call(
      body,
      grid=(nb,),
      in_specs=[
          pl.BlockSpec((bn, 2), lambda i: (i, 0)),
          pl.BlockSpec(emb1.shape, lambda i: (0, 0)),
          pl.BlockSpec(emb2.shape, lambda i: (0, 0)),
          pl.BlockSpec((NC, bn, hd), lambda i: (0, i, 0)),
          pl.BlockSpec((d, hd), lambda i: (0, 0)),
      ],
      out_specs=[
          pl.BlockSpec((bn, hd), lambda i: (i, 0)),
          pl.BlockSpec((bn, hd), lambda i: (i, 0)),
      ],
      out_shape=[
          jax.ShapeDtypeStruct((n, hd), jnp.float32),
          jax.ShapeDtypeStruct((n, hd), jnp.float32),
      ],
  )(x, emb1, emb2, eagg, w_edge)


def _tc_layer_mlp(h, s_part, eproj, w1, b1, w2, b2, *, bn, nb):
  def body(h_ref, s_ref, ep_ref, w1_ref, b1_ref, w2_ref, b2_ref, z_ref,
           st_ref):
    i = pl.program_id(0)
    hin = h_ref[...] + s_ref[0] + s_ref[1] + ep_ref[...]
    a = jnp.maximum(
        jnp.dot(hin.astype(jnp.bfloat16), w1_ref[...].astype(jnp.bfloat16),
                preferred_element_type=jnp.float32) + b1_ref[...], 0.0)
    z = jnp.dot(a.astype(jnp.bfloat16), w2_ref[...].astype(jnp.bfloat16),
                preferred_element_type=jnp.float32) + b2_ref[...]
    z_ref[...] = z

    @pl.when(i == 0)
    def _():
      st_ref[0:2, :] = jnp.zeros_like(st_ref[0:2, :])
      st_ref[2:3, :] = jnp.mean(z, axis=0, keepdims=True)

    m0 = st_ref[2:3, :]
    zc = z - m0
    st_ref[0:1, :] += jnp.sum(zc, axis=0, keepdims=True)
    st_ref[1:2, :] += jnp.sum(zc * zc, axis=0, keepdims=True)

  n, hd = h.shape
  h2 = w1.shape[1]
  return pl.pallas_call(
      body,
      grid=(nb,),
      in_specs=[
          pl.BlockSpec((bn, hd), lambda i: (i, 0)),
          pl.BlockSpec((NC, bn, hd), lambda i: (0, i, 0)),
          pl.BlockSpec((bn, hd), lambda i: (i, 0)),
          pl.BlockSpec((hd, h2), lambda i: (0, 0)),
          pl.BlockSpec((1, h2), lambda i: (0, 0)),
          pl.BlockSpec((h2, hd), lambda i: (0, 0)),
          pl.BlockSpec((1, hd), lambda i: (0, 0)),
      ],
      out_specs=[
          pl.BlockSpec((bn, hd), lambda i: (i, 0)),
          pl.BlockSpec((8, hd), lambda i: (0, 0)),
      ],
      out_shape=[
          jax.ShapeDtypeStruct((n, hd), jnp.float32),
          jax.ShapeDtypeStruct((8, hd), jnp.float32),
      ],
  )(h, s_part, eproj, w1, b1, w2, b2)


def _tc_bn(z, stats, gamma, beta, *, relu, n, bn, nb):
  inv_n = 1.0 / n

  def body(z_ref, st_ref, g_ref, b_ref, o_ref):
    dm = st_ref[0:1, :] * inv_n
    mean = st_ref[2:3, :] + dm
    var = st_ref[1:2, :] * inv_n - dm * dm
    rstd = lax.rsqrt(var + 1e-5)
    y = (z_ref[...] - mean) * (rstd * g_ref[...]) + b_ref[...]
    if relu:
      y = jnp.maximum(y, 0.0)
    o_ref[...] = y

  hd = z.shape[1]
  return pl.pallas_call(
      body,
      grid=(nb,),
      in_specs=[
          pl.BlockSpec((bn, hd), lambda i: (i, 0)),
          pl.BlockSpec((8, hd), lambda i: (0, 0)),
          pl.BlockSpec((1, hd), lambda i: (0, 0)),
          pl.BlockSpec((1, hd), lambda i: (0, 0)),
      ],
      out_specs=pl.BlockSpec((bn, hd), lambda i: (i, 0)),
      out_shape=jax.ShapeDtypeStruct(z.shape, jnp.float32),
  )(z, stats, gamma, beta)


def _tc_pool(h, batch2d, w_feat, b_feat, w_out_p, b_out_p, *, bn, nb):
  def body(h_ref, bt_ref, wf_ref, bf_ref, wo_ref, bo_ref, g_ref, l_ref,
           gsum, cnt):
    i = pl.program_id(0)

    @pl.when(i == 0)
    def _():
      gsum[...] = jnp.zeros_like(gsum)
      cnt[...] = jnp.zeros_like(cnt)

    oh = (lax.broadcasted_iota(jnp.int32, (bn, N_GRAPHS), 1)
          == bt_ref[...]).astype(jnp.float32)
    dn = (((0,), (0,)), ((), ()))
    gsum[...] += lax.dot_general(oh, h_ref[...], dn, precision=_HIGH,
                                 preferred_element_type=jnp.float32)
    cnt[...] += lax.dot_general(oh, jnp.ones_like(h_ref[...]), dn,
                                precision=_HIGH,
                                preferred_element_type=jnp.float32)

    @pl.when(i == nb - 1)
    def _():
      g = gsum[...] / jnp.maximum(cnt[...], 1.0)
      go = jnp.dot(g.astype(jnp.bfloat16), wf_ref[...].astype(jnp.bfloat16),
                   preferred_element_type=jnp.float32) + bf_ref[...]
      g_ref[...] = go
      l_ref[...] = jnp.dot(go.astype(jnp.bfloat16),
                           wo_ref[...].astype(jnp.bfloat16),
                           preferred_element_type=jnp.float32) + bo_ref[...]

  hd = h.shape[1]
  return pl.pallas_call(
      body,
      grid=(nb,),
      in_specs=[
          pl.BlockSpec((bn, hd), lambda i: (i, 0)),
          pl.BlockSpec((bn, 1), lambda i: (i, 0)),
          pl.BlockSpec((hd, hd), lambda i: (0, 0)),
          pl.BlockSpec((1, hd), lambda i: (0, 0)),
          pl.BlockSpec((hd, hd), lambda i: (0, 0)),
          pl.BlockSpec((1, hd), lambda i: (0, 0)),
      ],
      out_specs=[
          pl.BlockSpec((N_GRAPHS, hd), lambda i: (0, 0)),
          pl.BlockSpec((N_GRAPHS, hd), lambda i: (0, 0)),
      ],
      out_shape=[
          jax.ShapeDtypeStruct((N_GRAPHS, hd), jnp.float32),
          jax.ShapeDtypeStruct((N_GRAPHS, hd), jnp.float32),
      ],
      scratch_shapes=[
          pltpu.VMEM((N_GRAPHS, hd), jnp.float32),
          pltpu.VMEM((N_GRAPHS, hd), jnp.float32),
      ],
  )(h, batch2d, w_feat, b_feat, w_out_p, b_out_p)


def kernel(x, e_index, e_attr, batch, emb1, emb2, W_edge, W1, b1, W2, b2,
           bn_scale, bn_bias, W_feat, b_feat, W_out, b_out):
  n, hd = x.shape[0], emb1.shape[1]
  e = e_index.shape[1]
  d = e_attr.shape[1]
  num_layers = W1.shape[0]
  n_classes = W_out.shape[1]
  bn = 1000
  nb = n // bn

  chunks = SB * (-(-e // (NW * K * SB)))
  e_pad = NW * chunks * K
  pad = e_pad - e
  n_pad = NS * 8 * (-(-(n + 1) // (NS * 8)))

  src = e_index[0].astype(jnp.int32)
  dst = e_index[1].astype(jnp.int32)
  pad_src = jnp.arange(pad, dtype=jnp.int32) % n
  src3 = jnp.concatenate([src, pad_src]).reshape(NW, chunks, K)
  trash = n + jnp.arange(pad, dtype=jnp.int32) % (n_pad - n)
  dst3 = jnp.concatenate([dst, trash]).reshape(NW, chunks, K)
  ea_bf = e_attr.astype(jnp.bfloat16).astype(jnp.float32)
  ea_c = jnp.concatenate([ea_bf, jnp.zeros((pad, d), jnp.float32)]).reshape(
      e_pad * d // hd, hd)
  zeros_h = jnp.zeros((n_pad, hd), jnp.float32)

  eagg = _sc_eagg(ea_c, dst3, zeros_h, chunks=chunks, n_pad=n_pad, d=d,
                  h_dim=hd)
  h, eproj = _tc_prep(x.astype(jnp.int32), emb1, emb2, eagg, W_edge,
                      bn=bn, nb=nb)

  b1r = b1.reshape(num_layers, 1, -1)
  b2r = b2.reshape(num_layers, 1, -1)
  for l in range(num_layers):
    s_part = _sc_spmm(h, src3, dst3, zeros_h, chunks=chunks, n_pad=n_pad,
                      h_dim=hd)
    z, stats = _tc_layer_mlp(h, s_part, eproj, W1[l], b1r[l], W2[l], b2r[l],
                             bn=bn, nb=nb)
    h = _tc_bn(z, stats, bn_scale[l:l + 1], bn_bias[l:l + 1],
               relu=(l < num_layers - 1), n=n, bn=bn, nb=nb)

  w_out_p = jnp.zeros((hd, hd), jnp.float32).at[:, :n_classes].set(W_out)
  b_out_p = jnp.zeros((1, hd), jnp.float32).at[0, :n_classes].set(b_out)
  g, logits_p = _tc_pool(h, batch.astype(jnp.int32).reshape(n, 1),
                         W_feat, b_feat.reshape(1, hd), w_out_p, b_out_p,
                         bn=bn, nb=nb)
  return (g, logits_p[:, :n_classes])
